# baseline (speedup 1.0000x reference)
import functools

import jax
import jax.numpy as jnp
from jax import lax
from jax.experimental import pallas as pl
from jax.experimental.pallas import tpu as pltpu
from jax.experimental.pallas import tpu_sc as plsc

NUM_SEQ = 16384
SEQ = 50
DIM = 64
VOCAB = 1_000_000
VB = 4096
BT = 2048
NC = 2
NS = 16
NW = NC * NS
CHUNK = 128
BPW = NUM_SEQ // NW
G = BPW // CHUNK
NBUF = 2

_mesh = plsc.VectorSubcoreMesh(core_axis_name="c", subcore_axis_name="s")


@functools.partial(
    pl.kernel,
    mesh=_mesh,
    out_type=jax.ShapeDtypeStruct((SEQ, NUM_SEQ, DIM), jnp.float32),
    scratch_types=[
        pltpu.VMEM((SEQ, BPW), jnp.int32),
        pltpu.VMEM((NBUF, BPW, DIM), jnp.float32),
        pltpu.SemaphoreType.DMA((NBUF,)),
        pltpu.SemaphoreType.DMA((NBUF,)),
    ],
    compiler_params=pltpu.CompilerParams(use_tc_tiling_on_sc=False),
)
def _embed_kernel(idsT_hbm, table_hbm, out_hbm, idx_v, rows_v, sem_g, sem_w):
    wid = lax.axis_index("s") * NC + lax.axis_index("c")
    b0 = pl.multiple_of(wid * BPW, 128)
    pltpu.sync_copy(idsT_hbm.at[:, pl.ds(b0, BPW)], idx_v)

    def gather_descs(buf, s):
        return [
            pltpu.make_async_copy(
                table_hbm.at[idx_v.at[s, pl.ds(j * CHUNK, CHUNK)]],
                rows_v.at[buf, pl.ds(j * CHUNK, CHUNK)],
                sem_g.at[buf],
            )
            for j in range(G)
        ]

    def write_desc(buf, s):
        return pltpu.make_async_copy(
            rows_v.at[buf], out_hbm.at[s, pl.ds(b0, BPW)], sem_w.at[buf]
        )

    def start_gathers(buf, s):
        for d in gather_descs(buf, s):
            d.start()

    def step(buf, s):
        for d in gather_descs(buf, s):
            d.wait()
        write_desc(buf, s).start()
        write_desc(buf, s).wait()

        @pl.when(s + NBUF < SEQ)
        def _():
            start_gathers(buf, s + NBUF)

    for b in range(NBUF):
        start_gathers(b, b)

    def body(t, carry):
        for b in range(NBUF):
            step(b, t * NBUF + b)
        return carry

    lax.fori_loop(0, SEQ // NBUF, body, 0)


def _repack_body(wt_ref, out_ref):
    out_ref[...] = wt_ref[...].T


_repack = pl.pallas_call(
    _repack_body,
    grid=(pl.cdiv(VOCAB, VB),),
    in_specs=[pl.BlockSpec((DIM, VB), lambda i: (0, i))],
    out_specs=pl.BlockSpec((VB, DIM), lambda i: (i, 0)),
    out_shape=jax.ShapeDtypeStruct((VOCAB, DIM), jnp.float32),
)


def _tx_body(in_ref, out_ref):
    x = in_ref[...].reshape(BT, DIM)
    out_ref[...] = x.T.reshape(1, DIM, BT)


_tx = pl.pallas_call(
    _tx_body,
    grid=(SEQ, NUM_SEQ // BT),
    in_specs=[pl.BlockSpec((1, BT, DIM), lambda s, b: (s, b, 0))],
    out_specs=pl.BlockSpec((1, DIM, BT), lambda s, b: (s, 0, b)),
    out_shape=jax.ShapeDtypeStruct((SEQ, DIM, NUM_SEQ), jnp.float32),
)


def kernel(input_ids, weight):
    table = _repack(weight.T)
    rm = _embed_kernel(input_ids.T.astype(jnp.int32), table)
    outT = _tx(rm)
    return outT.transpose(2, 0, 1)


# speedup vs baseline: 1.1291x; 1.1291x over previous
import functools

import jax
import jax.numpy as jnp
from jax import lax
from jax.experimental import pallas as pl
from jax.experimental.pallas import tpu as pltpu
from jax.experimental.pallas import tpu_sc as plsc

NUM_SEQ = 16384
SEQ = 50
DIM = 64
VOCAB = 1_000_000
VB = 8192
BT = 4096
NC = 2
NS = 16
NW = NC * NS
CHUNK = 128
BPW = NUM_SEQ // NW
G = BPW // CHUNK
NBUF = 2

_mesh = plsc.VectorSubcoreMesh(core_axis_name="c", subcore_axis_name="s")


@functools.partial(
    pl.kernel,
    mesh=_mesh,
    out_type=jax.ShapeDtypeStruct((SEQ, NUM_SEQ, DIM), jnp.float32),
    scratch_types=[
        pltpu.VMEM((SEQ, BPW), jnp.int32),
        pltpu.VMEM((NBUF, BPW, DIM), jnp.float32),
        pltpu.SemaphoreType.DMA((NBUF,)),
        pltpu.SemaphoreType.DMA((NBUF,)),
    ],
    compiler_params=pltpu.CompilerParams(use_tc_tiling_on_sc=False),
)
def _embed_kernel(idsT_hbm, table_hbm, out_hbm, idx_v, rows_v, sem_g, sem_w):
    wid = lax.axis_index("s") * NC + lax.axis_index("c")
    b0 = pl.multiple_of(wid * BPW, 128)
    pltpu.sync_copy(idsT_hbm.at[:, pl.ds(b0, BPW)], idx_v)

    def gather_descs(buf, s):
        return [
            pltpu.make_async_copy(
                table_hbm.at[idx_v.at[s, pl.ds(j * CHUNK, CHUNK)]],
                rows_v.at[buf, pl.ds(j * CHUNK, CHUNK)],
                sem_g.at[buf],
            )
            for j in range(G)
        ]

    def write_desc(buf, s):
        return pltpu.make_async_copy(
            rows_v.at[buf], out_hbm.at[s, pl.ds(b0, BPW)], sem_w.at[buf]
        )

    def start_gathers(buf, s):
        for d in gather_descs(buf, s):
            d.start()

    def step(buf, s):
        for d in gather_descs(buf, s):
            d.wait()
        write_desc(buf, s).start()
        write_desc(buf, s).wait()

        @pl.when(s + NBUF < SEQ)
        def _():
            start_gathers(buf, s + NBUF)

    for b in range(NBUF):
        start_gathers(b, b)

    def body(t, carry):
        for b in range(NBUF):
            step(b, t * NBUF + b)
        return carry

    lax.fori_loop(0, SEQ // NBUF, body, 0)


def _repack_body(wt_ref, out_ref):
    x = wt_ref[...]
    eye = jnp.eye(DIM, dtype=jnp.float32)
    out_ref[...] = jax.lax.dot_general(
        x, eye, (((0,), (0,)), ((), ())),
        preferred_element_type=jnp.float32,
    )


_repack = pl.pallas_call(
    _repack_body,
    grid=(pl.cdiv(VOCAB, VB),),
    in_specs=[pl.BlockSpec((DIM, VB), lambda i: (0, i))],
    out_specs=pl.BlockSpec((VB, DIM), lambda i: (i, 0)),
    out_shape=jax.ShapeDtypeStruct((VOCAB, DIM), jnp.float32),
)


def _tx_body(in_ref, out_ref):
    x = in_ref[...].reshape(BT, DIM)
    eye = jnp.eye(DIM, dtype=jnp.float32)
    xt = jax.lax.dot_general(
        eye, x, (((1,), (1,)), ((), ())),
        preferred_element_type=jnp.float32,
    )
    out_ref[...] = xt.reshape(1, DIM, BT)


_tx = pl.pallas_call(
    _tx_body,
    grid=(SEQ, NUM_SEQ // BT),
    in_specs=[pl.BlockSpec((1, BT, DIM), lambda s, b: (s, b, 0))],
    out_specs=pl.BlockSpec((1, DIM, BT), lambda s, b: (s, 0, b)),
    out_shape=jax.ShapeDtypeStruct((SEQ, DIM, NUM_SEQ), jnp.float32),
)


def kernel(input_ids, weight):
    table = _repack(weight.T)
    rm = _embed_kernel(input_ids.T.astype(jnp.int32), table)
    outT = _tx(rm)
    return outT.transpose(2, 0, 1)
